# baseline (speedup 1.0000x reference)
import jax
import jax.numpy as jnp
from jax.experimental import pallas as pl

N = 10000
G = 64


def _normalize_body(pooled_ref, out_ref):
    p = pooled_ref[...]
    nrm = jnp.sqrt(jnp.sum(p * p, axis=1, keepdims=True))
    out_ref[...] = p / jnp.maximum(nrm, 1e-12)


def _gcn_conv(x, src, dst, W, b, n):
    h = x @ W
    deg = jnp.zeros((n,), dtype=x.dtype).at[dst].add(1.0)
    dinv = jnp.where(deg > 0, 1.0 / jnp.sqrt(deg), 0.0)
    norm = dinv[src] * dinv[dst]
    msgs = h[src] * norm[:, None]
    out = jnp.zeros((n, h.shape[1]), dtype=x.dtype).at[dst].add(msgs)
    return out + b


def kernel(x, edge_index, batch, W1, b1, W2, b2, W3, b3):
    loops = jnp.arange(N, dtype=edge_index.dtype)
    src = jnp.concatenate([edge_index[0], loops])
    dst = jnp.concatenate([edge_index[1], loops])
    h = jax.nn.relu(_gcn_conv(x, src, dst, W1, b1, N))
    h = jax.nn.relu(_gcn_conv(h, src, dst, W2, b2, N))
    h = _gcn_conv(h, src, dst, W3, b3, N)
    cnt = jax.ops.segment_sum(jnp.ones((N,), dtype=h.dtype), batch, num_segments=G)
    summ = jax.ops.segment_sum(h, batch, num_segments=G)
    pooled = summ / jnp.maximum(cnt, 1.0)[:, None]
    return pl.pallas_call(
        _normalize_body,
        out_shape=jax.ShapeDtypeStruct((G, 128), jnp.float32),
    )(pooled)


# speedup vs baseline: 4.7263x; 4.7263x over previous
import functools

import jax
import jax.numpy as jnp
from jax import lax
from jax.experimental import pallas as pl
from jax.experimental.pallas import tpu as pltpu
from jax.experimental.pallas import tpu_sc as plsc

N = 10000
E = 320000
F_IN = 128
H = 256
D_EMB = 128
G = 64

NPAD = 10240
NC = 2
NS = 16
NW = NC * NS
EPAD = 327680
EPW = EPAD // NW
B = 128
NB = EPW // B
RPT = NPAD // NS

_mesh = plsc.VectorSubcoreMesh(
    core_axis_name="c", subcore_axis_name="s", num_cores=NC, num_subcores=NS)


def _deg_body(dst_hbm, zeros_hbm, ones_hbm, out_hbm, didx, ones_v, acc, sem):
    c = lax.axis_index("c")
    s = lax.axis_index("s")
    wid = s * NC + c
    pltpu.sync_copy(zeros_hbm.at[pl.ds(s * RPT, RPT)], acc.at[pl.ds(s * RPT, RPT)])
    pltpu.sync_copy(ones_hbm, ones_v)
    plsc.subcore_barrier()

    def step(i, carry):
        off = wid * EPW + i * B
        pltpu.sync_copy(dst_hbm.at[pl.ds(off, B)], didx)
        pltpu.sync_copy(ones_v, acc.at[didx], add=True)
        return carry

    lax.fori_loop(0, NB, step, 0)
    plsc.subcore_barrier()
    pltpu.sync_copy(acc.at[pl.ds(s * RPT, RPT)], out_hbm.at[c, pl.ds(s * RPT, RPT)])


_deg_kernel = functools.partial(
    pl.kernel,
    out_type=jax.ShapeDtypeStruct((NC, NPAD, 128), jnp.float32),
    mesh=_mesh,
    scratch_types=[
        pltpu.VMEM((B,), jnp.int32),
        pltpu.VMEM((B, 128), jnp.float32),
        pltpu.VMEM_SHARED((NPAD, 128), jnp.float32),
        pltpu.SemaphoreType.DMA,
    ],
)(_deg_body)


def _agg_body(src_hbm, dst_hbm, p_hbm, zeros_hbm, out_hbm, sidx, didx, rows, acc, sem):
    c = lax.axis_index("c")
    s = lax.axis_index("s")
    wid = s * NC + c
    pltpu.sync_copy(zeros_hbm.at[pl.ds(s * RPT, RPT)], acc.at[pl.ds(s * RPT, RPT)])
    plsc.subcore_barrier()

    def step(i, carry):
        off = wid * EPW + i * B
        pltpu.sync_copy(src_hbm.at[pl.ds(off, B)], sidx)
        pltpu.sync_copy(dst_hbm.at[pl.ds(off, B)], didx)
        pltpu.async_copy(p_hbm.at[sidx], rows, sem).wait()
        pltpu.sync_copy(rows, acc.at[didx], add=True)
        return carry

    lax.fori_loop(0, NB, step, 0)
    plsc.subcore_barrier()
    pltpu.sync_copy(acc.at[pl.ds(s * RPT, RPT)], out_hbm.at[c, pl.ds(s * RPT, RPT)])


_agg_kernel = functools.partial(
    pl.kernel,
    out_type=jax.ShapeDtypeStruct((NC, NPAD, 128), jnp.float32),
    mesh=_mesh,
    scratch_types=[
        pltpu.VMEM((B,), jnp.int32),
        pltpu.VMEM((B,), jnp.int32),
        pltpu.VMEM((B, 128), jnp.float32),
        pltpu.VMEM_SHARED((NPAD, 128), jnp.float32),
        pltpu.SemaphoreType.DMA,
    ],
)(_agg_body)


def _prep_body(deg_ref, x_ref, w1_ref, dinv_ref, p0_ref, p1_ref):
    degsum = deg_ref[0] + deg_ref[1]
    deg = degsum[:, 0:1] + 1.0
    dinv = lax.rsqrt(deg)
    dinv_ref[...] = jnp.broadcast_to(dinv, (_RB, 128))
    h = jnp.dot(x_ref[...], w1_ref[...], preferred_element_type=jnp.float32)
    h = h * dinv
    p0_ref[...] = h[:, :128]
    p1_ref[...] = h[:, 128:]


def _prep(deg, x_pad, W1):
    row_spec = pl.BlockSpec((_RB, 128), lambda i: (i, 0))
    return pl.pallas_call(
        _prep_body,
        grid=(NPAD // _RB,),
        in_specs=[
            pl.BlockSpec((NC, _RB, 128), lambda i: (0, i, 0)),
            row_spec,
            pl.BlockSpec((F_IN, H), lambda i: (0, 0)),
        ],
        out_specs=[row_spec, row_spec, row_spec],
        out_shape=[
            jax.ShapeDtypeStruct((NPAD, 128), jnp.float32),
            jax.ShapeDtypeStruct((NPAD, 128), jnp.float32),
            jax.ShapeDtypeStruct((NPAD, 128), jnp.float32),
        ],
    )(deg, x_pad, W1)


_RB = 2560


def _combine2_body(a0_ref, a1_ref, p0_ref, p1_ref, dinv_ref, b_ref, w_ref,
                   q0_ref, q1_ref):
    dinv = dinv_ref[...]
    h0 = (a0_ref[0] + a0_ref[1] + p0_ref[...]) * dinv
    h1 = (a1_ref[0] + a1_ref[1] + p1_ref[...]) * dinv
    h = jnp.concatenate([h0, h1], axis=1) + b_ref[...]
    h = jnp.maximum(h, 0.0)
    q = jnp.dot(h, w_ref[...], preferred_element_type=jnp.float32) * dinv[:, 0:1]
    q0_ref[...] = q[:, :128]
    q1_ref[...] = q[:, 128:]


def _combine2(a0, a1, p0, p1, dinv_b, b_vec, W):
    grid = (NPAD // _RB,)
    acc_spec = pl.BlockSpec((NC, _RB, 128), lambda i: (0, i, 0))
    row_spec = pl.BlockSpec((_RB, 128), lambda i: (i, 0))
    return pl.pallas_call(
        _combine2_body,
        grid=grid,
        in_specs=[
            acc_spec, acc_spec, row_spec, row_spec, row_spec,
            pl.BlockSpec((1, H), lambda i: (0, 0)),
            pl.BlockSpec((H, H), lambda i: (0, 0)),
        ],
        out_specs=[row_spec, row_spec],
        out_shape=[
            jax.ShapeDtypeStruct((NPAD, 128), jnp.float32),
            jax.ShapeDtypeStruct((NPAD, 128), jnp.float32),
        ],
    )(a0, a1, p0, p1, dinv_b, b_vec, W)


def _combine3_body(a0_ref, a1_ref, p0_ref, p1_ref, dinv_ref, b_ref, w_ref, q_ref):
    dinv = dinv_ref[...]
    h0 = (a0_ref[0] + a0_ref[1] + p0_ref[...]) * dinv
    h1 = (a1_ref[0] + a1_ref[1] + p1_ref[...]) * dinv
    h = jnp.concatenate([h0, h1], axis=1) + b_ref[...]
    h = jnp.maximum(h, 0.0)
    q_ref[...] = jnp.dot(h, w_ref[...], preferred_element_type=jnp.float32) * dinv


def _combine3(a0, a1, p0, p1, dinv_b, b_vec, W):
    grid = (NPAD // _RB,)
    acc_spec = pl.BlockSpec((NC, _RB, 128), lambda i: (0, i, 0))
    row_spec = pl.BlockSpec((_RB, 128), lambda i: (i, 0))
    return pl.pallas_call(
        _combine3_body,
        grid=grid,
        in_specs=[
            acc_spec, acc_spec, row_spec, row_spec, row_spec,
            pl.BlockSpec((1, H), lambda i: (0, 0)),
            pl.BlockSpec((H, D_EMB), lambda i: (0, 0)),
        ],
        out_specs=[row_spec],
        out_shape=[jax.ShapeDtypeStruct((NPAD, 128), jnp.float32)],
    )(a0, a1, p0, p1, dinv_b, b_vec, W)[0]


def _final_body(a_ref, p_ref, dinv_ref, b_ref, batch_ref, out_ref):
    h = (a_ref[0] + a_ref[1] + p_ref[...]) * dinv_ref[...] + b_ref[...]
    bvec = batch_ref[...]
    seg = lax.broadcasted_iota(jnp.int32, (G, NPAD), 0)
    m = (jnp.broadcast_to(bvec, (G, NPAD)) == seg).astype(jnp.float32)
    summ = jnp.dot(m, h, preferred_element_type=jnp.float32)
    cnt = jnp.sum(m, axis=1, keepdims=True)
    pooled = summ / jnp.maximum(cnt, 1.0)
    nrm = jnp.sqrt(jnp.sum(pooled * pooled, axis=1, keepdims=True))
    out_ref[...] = pooled / jnp.maximum(nrm, 1e-12)


def _final(a, p, dinv_b, b_vec, batch_2d):
    return pl.pallas_call(
        _final_body,
        out_shape=jax.ShapeDtypeStruct((G, D_EMB), jnp.float32),
    )(a, p, dinv_b, b_vec, batch_2d)


def kernel(x, edge_index, batch, W1, b1, W2, b2, W3, b3):
    f32 = jnp.float32
    i32 = jnp.int32
    pad_e = EPAD - E
    src = jnp.concatenate([edge_index[0], jnp.full((pad_e,), NPAD - 1, i32)])
    dst = jnp.concatenate([edge_index[1], jnp.full((pad_e,), NPAD - 1, i32)])
    x_pad = jnp.concatenate([x, jnp.zeros((NPAD - N, F_IN), f32)], axis=0)
    batch_2d = jnp.concatenate([batch, jnp.full((NPAD - N,), G, i32)])[None, :]
    ones128 = jnp.ones((B, 128), f32)
    zeros128 = jnp.zeros((NPAD, 128), f32)

    deg = _deg_kernel(dst, zeros128, ones128)
    dinv_b, p0, p1 = _prep(deg, x_pad, W1)

    a0 = _agg_kernel(src, dst, p0, zeros128)
    a1 = _agg_kernel(src, dst, p1, zeros128)
    q0, q1 = _combine2(a0, a1, p0, p1, dinv_b, b1[None, :], W2)

    a0 = _agg_kernel(src, dst, q0, zeros128)
    a1 = _agg_kernel(src, dst, q1, zeros128)
    r0 = _combine3(a0, a1, q0, q1, dinv_b, b2[None, :], W3)

    a = _agg_kernel(src, dst, r0, zeros128)
    return _final(a, r0, dinv_b, b3[None, :], batch_2d)
